# baseline (speedup 1.0000x reference)
import jax
import jax.numpy as jnp
from jax.experimental import pallas as pl
from jax.experimental.pallas import tpu as pltpu

_STRIDE = 32.0
_G = 19
_GG = _G * _G
_NA = 3
_NC = 85
_LANES = 128
_C = 1024
_AW = (116.0, 156.0, 373.0)
_AH = (90.0, 198.0, 326.0)

_NSLOT = 3
_CH = 512


def _yolo_body(x_hbm, w_ref, b_ref, o_hbm, x_vmem, o_vmem, w_bf, mm_vmem,
               in_sem, out_sem):
    i = pl.program_id(0)
    nb = pl.num_programs(0) - 1
    slot = jax.lax.rem(i, _NSLOT)

    def copy_in(b, s, h):
        return pltpu.make_async_copy(
            x_hbm.at[:, b, h * _CH:(h + 1) * _CH],
            x_vmem.at[s, :, h * _CH:(h + 1) * _CH],
            in_sem.at[s, h])

    def copy_out(b, s):
        return pltpu.make_async_copy(
            o_vmem.at[s], o_hbm.at[:, b, :], out_sem.at[s])

    @pl.when(i == 0)
    def _():
        for h in range(2):
            copy_in(0, 0, h).start()
            copy_in(1, 1, h).start()
        w1024 = w_ref[...].reshape(_NA * _NC, 8, 128).reshape(_NA * _NC, _C)
        for k in range(_NA):
            w_bf[k * _LANES:k * _LANES + _NC, :] = (
                w1024[k * _NC:(k + 1) * _NC, :].astype(jnp.bfloat16))

    @pl.when(i + 2 < nb)
    def _():
        for h in range(2):
            copy_in(i + 2, jax.lax.rem(i + 2, _NSLOT), h).start()

    @pl.when(i < nb)
    def _():
        for h in range(2):
            copy_in(i, slot, h).wait()
        x = x_vmem[slot].astype(jnp.bfloat16)
        w = w_bf[...]
        mm_vmem[jax.lax.rem(i, 2)] = jax.lax.dot_general(
            w, x, (((1,), (1,)), ((), ())),
            preferred_element_type=jnp.float32)

    @pl.when(i >= 1)
    def _():
        d_i = i - 1
        oslot = jax.lax.rem(d_i, _NSLOT)
        @pl.when(d_i >= _NSLOT)
        def _():
            copy_out(d_i - _NSLOT, oslot).wait()

        mm = mm_vmem[jax.lax.rem(d_i, 2)] + b_ref[...]
        lane = jax.lax.broadcasted_iota(jnp.int32, (_NC, _GG), 1)
        row = jax.lax.broadcasted_iota(jnp.int32, (_NC, _GG), 0)
        gy = (lane // _G).astype(jnp.float32)
        gx = (lane % _G).astype(jnp.float32)
        for k in range(_NA):
            s = mm[k * _LANES:k * _LANES + _NC, :]
            sig = jax.nn.sigmoid(s)
            bw = jnp.broadcast_to(jnp.exp(s[2:3, :]) * _AW[k], s.shape)
            bh = jnp.broadcast_to(jnp.exp(s[3:4, :]) * _AH[k], s.shape)
            val = jnp.where(row == 0, (sig + gx) * _STRIDE,
                  jnp.where(row == 1, (sig + gy) * _STRIDE,
                  jnp.where(row == 2, bw,
                  jnp.where(row == 3, bh, sig))))
            o_vmem[oslot, :, k * _GG:(k + 1) * _GG] = val
        copy_out(d_i, oslot).start()

        @pl.when(d_i == nb - 1)
        def _():
            for d in range(_NSLOT - 1):
                copy_out(d_i - 1 - d,
                         jax.lax.rem(d_i - 1 - d, _NSLOT)).wait()
            copy_out(d_i, oslot).wait()


def kernel(xin, W, b):
    B, C, G, _ = xin.shape
    xp = xin.transpose(2, 3, 0, 1).reshape(_GG, B, C)
    wv = W.reshape(_NA * _NC * 8, _LANES)
    bp = jnp.pad(b.reshape(_NA, _NC), ((0, 0), (0, _LANES - _NC)))
    bp = bp.reshape(_NA * _LANES, 1)
    out = pl.pallas_call(
        _yolo_body,
        grid=(B + 1,),
        in_specs=[
            pl.BlockSpec(memory_space=pltpu.MemorySpace.HBM),
            pl.BlockSpec((_NA * _NC * 8, _LANES), lambda i: (0, 0)),
            pl.BlockSpec((_NA * _LANES, 1), lambda i: (0, 0)),
        ],
        out_specs=pl.BlockSpec(memory_space=pltpu.MemorySpace.HBM),
        out_shape=jax.ShapeDtypeStruct((_NC, B, _NA * _GG), jnp.float32),
        scratch_shapes=[
            pltpu.VMEM((_NSLOT, _GG, _C), jnp.float32),
            pltpu.VMEM((_NSLOT, _NC, _NA * _GG), jnp.float32),
            pltpu.VMEM((_NA * _LANES, _C), jnp.bfloat16),
            pltpu.VMEM((2, _NA * _LANES, _GG), jnp.float32),
            pltpu.SemaphoreType.DMA((_NSLOT, 2)),
            pltpu.SemaphoreType.DMA((_NSLOT,)),
        ],
    )(xp, wv, bp)
    return out.transpose(1, 2, 0)


# speedup vs baseline: 1.0756x; 1.0756x over previous
import jax
import jax.numpy as jnp
from jax.experimental import pallas as pl
from jax.experimental.pallas import tpu as pltpu

_STRIDE = 32.0
_G = 19
_GG = _G * _G
_NA = 3
_NC = 85
_LANES = 128
_C = 1024
_AW = (116.0, 156.0, 373.0)
_AH = (90.0, 198.0, 326.0)

_NSLOT = 3
_CH = 256


def _yolo_body(x_hbm, w_ref, b_ref, o_hbm, x_vmem, o_vmem, w_bf,
               in_sem, out_sem):
    i = pl.program_id(0)
    nb = pl.num_programs(0)
    slot = jax.lax.rem(i, _NSLOT)

    def copy_in(b, s, h):
        return pltpu.make_async_copy(
            x_hbm.at[:, b, h * _CH:(h + 1) * _CH],
            x_vmem.at[s, :, h * _CH:(h + 1) * _CH],
            in_sem.at[s, h])

    def copy_out(b, s):
        return pltpu.make_async_copy(
            o_vmem.at[s], o_hbm.at[:, b, :], out_sem.at[s])

    @pl.when(i == 0)
    def _():
        for h in range(4):
            copy_in(0, 0, h).start()
            copy_in(1, 1, h).start()
        w1024 = w_ref[...].reshape(_NA * _NC, 8, 128).reshape(_NA * _NC, _C)
        for k in range(_NA):
            w_bf[k * _LANES:k * _LANES + _NC, :] = (
                w1024[k * _NC:(k + 1) * _NC, :].astype(jnp.bfloat16))

    @pl.when(i + 2 < nb)
    def _():
        for h in range(4):
            copy_in(i + 2, jax.lax.rem(i + 2, _NSLOT), h).start()

    for h in range(4):
        copy_in(i, slot, h).wait()
    x = x_vmem[slot].astype(jnp.bfloat16)
    w = w_bf[...]
    mm = jax.lax.dot_general(
        w, x, (((1,), (1,)), ((), ())),
        preferred_element_type=jnp.float32)
    mm = mm + b_ref[...]

    @pl.when(i >= _NSLOT)
    def _():
        copy_out(i - _NSLOT, slot).wait()

    lane = jax.lax.broadcasted_iota(jnp.int32, (_NC, _GG), 1)
    row = jax.lax.broadcasted_iota(jnp.int32, (_NC, _GG), 0)
    gy = (lane // _G).astype(jnp.float32)
    gx = (lane % _G).astype(jnp.float32)
    for k in range(_NA):
        s = mm[k * _LANES:k * _LANES + _NC, :]
        sig = jax.nn.sigmoid(s)
        bw = jnp.broadcast_to(jnp.exp(s[2:3, :]) * _AW[k], s.shape)
        bh = jnp.broadcast_to(jnp.exp(s[3:4, :]) * _AH[k], s.shape)
        val = jnp.where(row == 0, (sig + gx) * _STRIDE,
              jnp.where(row == 1, (sig + gy) * _STRIDE,
              jnp.where(row == 2, bw,
              jnp.where(row == 3, bh, sig))))
        o_vmem[slot, :, k * _GG:(k + 1) * _GG] = val
    copy_out(i, slot).start()

    @pl.when(i == nb - 1)
    def _():
        for d in range(_NSLOT - 1):
            copy_out(i - 1 - d, jax.lax.rem(i - 1 - d, _NSLOT)).wait()
        copy_out(i, slot).wait()


def kernel(xin, W, b):
    B, C, G, _ = xin.shape
    xp = xin.transpose(2, 3, 0, 1).reshape(_GG, B, C)
    wv = W.reshape(_NA * _NC * 8, _LANES)
    bp = jnp.pad(b.reshape(_NA, _NC), ((0, 0), (0, _LANES - _NC)))
    bp = bp.reshape(_NA * _LANES, 1)
    out = pl.pallas_call(
        _yolo_body,
        grid=(B,),
        in_specs=[
            pl.BlockSpec(memory_space=pltpu.MemorySpace.HBM),
            pl.BlockSpec((_NA * _NC * 8, _LANES), lambda i: (0, 0)),
            pl.BlockSpec((_NA * _LANES, 1), lambda i: (0, 0)),
        ],
        out_specs=pl.BlockSpec(memory_space=pltpu.MemorySpace.HBM),
        out_shape=jax.ShapeDtypeStruct((_NC, B, _NA * _GG), jnp.float32),
        scratch_shapes=[
            pltpu.VMEM((_NSLOT, _GG, _C), jnp.float32),
            pltpu.VMEM((_NSLOT, _NC, _NA * _GG), jnp.float32),
            pltpu.VMEM((_NA * _LANES, _C), jnp.bfloat16),
            pltpu.SemaphoreType.DMA((_NSLOT, 4)),
            pltpu.SemaphoreType.DMA((_NSLOT,)),
        ],
    )(xp, wv, bp)
    return out.transpose(1, 2, 0)


# speedup vs baseline: 1.0933x; 1.0165x over previous
import jax
import jax.numpy as jnp
from jax.experimental import pallas as pl
from jax.experimental.pallas import tpu as pltpu

_STRIDE = 32.0
_G = 19
_GG = _G * _G
_NA = 3
_NC = 85
_LANES = 128
_C = 1024
_AW = (116.0, 156.0, 373.0)
_AH = (90.0, 198.0, 326.0)

_NSLOT = 3
_CH = 512


def _yolo_body(x_hbm, w_ref, b_ref, o_hbm, x_vmem, o_vmem, w_bf,
               in_sem, out_sem):
    i = pl.program_id(0)
    nb = pl.num_programs(0)
    slot = jax.lax.rem(i, _NSLOT)

    def copy_in(b, s, h):
        return pltpu.make_async_copy(
            x_hbm.at[:, b, h * _CH:(h + 1) * _CH],
            x_vmem.at[s, :, h * _CH:(h + 1) * _CH],
            in_sem.at[s, h])

    def copy_out(b, s):
        return pltpu.make_async_copy(
            o_vmem.at[s], o_hbm.at[:, b, :], out_sem.at[s])

    @pl.when(i == 0)
    def _():
        for h in range(2):
            copy_in(0, 0, h).start()
            copy_in(1, 1, h).start()
        w1024 = w_ref[...].reshape(_NA * _NC, 8, 128).reshape(_NA * _NC, _C)
        for k in range(_NA):
            w_bf[k * _LANES:k * _LANES + _NC, :] = (
                w1024[k * _NC:(k + 1) * _NC, :].astype(jnp.bfloat16))

    @pl.when(i + 2 < nb)
    def _():
        for h in range(2):
            copy_in(i + 2, jax.lax.rem(i + 2, _NSLOT), h).start()

    for h in range(2):
        copy_in(i, slot, h).wait()
    x = x_vmem[slot].astype(jnp.bfloat16)
    w = w_bf[...]
    mm = jax.lax.dot_general(
        w, x, (((1,), (1,)), ((), ())),
        preferred_element_type=jnp.float32)
    mm = mm + b_ref[...]

    @pl.when(i >= _NSLOT)
    def _():
        copy_out(i - _NSLOT, slot).wait()

    lane = jax.lax.broadcasted_iota(jnp.int32, (_NC, _GG), 1)
    row = jax.lax.broadcasted_iota(jnp.int32, (_NC, _GG), 0)
    gy = (lane // _G).astype(jnp.float32)
    gx = (lane % _G).astype(jnp.float32)
    for k in range(_NA):
        s = mm[k * _LANES:k * _LANES + _NC, :]
        sig = jax.nn.sigmoid(s)
        bw = jnp.broadcast_to(jnp.exp(s[2:3, :]) * _AW[k], s.shape)
        bh = jnp.broadcast_to(jnp.exp(s[3:4, :]) * _AH[k], s.shape)
        val = jnp.where(row == 0, (sig + gx) * _STRIDE,
              jnp.where(row == 1, (sig + gy) * _STRIDE,
              jnp.where(row == 2, bw,
              jnp.where(row == 3, bh, sig))))
        o_vmem[slot, :, k * _GG:(k + 1) * _GG] = val
    copy_out(i, slot).start()

    @pl.when(i == nb - 1)
    def _():
        for d in range(_NSLOT - 1):
            copy_out(i - 1 - d, jax.lax.rem(i - 1 - d, _NSLOT)).wait()
        copy_out(i, slot).wait()


def kernel(xin, W, b):
    B, C, G, _ = xin.shape
    xp = xin.transpose(2, 3, 0, 1).reshape(_GG, B, C)
    wv = W.reshape(_NA * _NC * 8, _LANES)
    bp = jnp.pad(b.reshape(_NA, _NC), ((0, 0), (0, _LANES - _NC)))
    bp = bp.reshape(_NA * _LANES, 1)
    out = pl.pallas_call(
        _yolo_body,
        grid=(B,),
        in_specs=[
            pl.BlockSpec(memory_space=pltpu.MemorySpace.HBM),
            pl.BlockSpec((_NA * _NC * 8, _LANES), lambda i: (0, 0)),
            pl.BlockSpec((_NA * _LANES, 1), lambda i: (0, 0)),
        ],
        out_specs=pl.BlockSpec(memory_space=pltpu.MemorySpace.HBM),
        out_shape=jax.ShapeDtypeStruct((_NC, B, _NA * _GG), jnp.float32),
        scratch_shapes=[
            pltpu.VMEM((_NSLOT, _GG, _C), jnp.float32),
            pltpu.VMEM((_NSLOT, _NC, _NA * _GG), jnp.float32),
            pltpu.VMEM((_NA * _LANES, _C), jnp.bfloat16),
            pltpu.SemaphoreType.DMA((_NSLOT, 2)),
            pltpu.SemaphoreType.DMA((_NSLOT,)),
        ],
    )(xp, wv, bp)
    return out.transpose(1, 2, 0)
